# baseline (device time: 60417 ns/iter reference)
import jax
import jax.numpy as jnp
from jax import lax
from jax.experimental import pallas as pl
from jax.experimental.pallas import tpu as pltpu

N_DEV = 8
B = 2
SQ = 512
DMODEL = 768
HQ = 8
DH = 64
DF = HQ * DH
PACK = DF + 8
SKV_SHARD = 512
SCALE = 0.125
NEG = -1e9
ROUNDS = (1, 3, 4)
NC = 8
CH = SQ // NC


def kernel(x, Wq, K_ext, V_ext, Wo):
    K2 = K_ext.reshape(B, SKV_SHARD, DF)
    V2 = V_ext.reshape(B, SKV_SHARD, DF)

    def body(x_ref, wq_ref, k_ref, v_ref, wo_ref, out_ref,
             comm_send, comm_recv, send_sems, recv_sems):
        my = lax.axis_index("i")

        barrier_sem = pltpu.get_barrier_semaphore()
        for xr in ROUNDS:
            pl.semaphore_signal(
                barrier_sem, inc=1,
                device_id=(my ^ xr,), device_id_type=pl.DeviceIdType.MESH,
            )
        pl.semaphore_wait(barrier_sem, 3)

        kj = lax.broadcasted_iota(jnp.int32, (CH, SKV_SHARD), 1) + my * SKV_SHARD

        L = [[[None] * HQ for _ in range(B)] for _ in range(NC)]
        O = [[[None] * HQ for _ in range(B)] for _ in range(NC)]
        rdmas = [[None] * NC for _ in range(len(ROUNDS))]

        def pack(c, b, h):
            comm_send[c, b, :, h * DH:(h + 1) * DH] = O[c][b][h].astype(jnp.bfloat16)
            comm_send[c, b, :, DF + h:DF + h + 1] = L[c][b][h].astype(jnp.bfloat16)

        def start_round(r, c):
            rdma = pltpu.make_async_remote_copy(
                src_ref=comm_send.at[c],
                dst_ref=comm_recv.at[r, c],
                send_sem=send_sems.at[r, c],
                recv_sem=recv_sems.at[r, c],
                device_id=(my ^ ROUNDS[r],),
                device_id_type=pl.DeviceIdType.MESH,
            )
            rdma.start()
            rdmas[r][c] = rdma

        for c in range(NC):
            qi = lax.broadcasted_iota(jnp.int32, (CH, SKV_SHARD), 0) + c * CH
            mask = (jnp.abs(qi - kj) <= 128) | (kj < 32) | (qi < 32)
            for b in range(B):
                q_bc = jnp.dot(x_ref[b, c * CH:(c + 1) * CH, :], wq_ref[:, :],
                               preferred_element_type=jnp.float32)
                for h in range(HQ):
                    q_bh = q_bc[:, h * DH:(h + 1) * DH]
                    k_bh = k_ref[b, :, h * DH:(h + 1) * DH]
                    v_bh = v_ref[b, :, h * DH:(h + 1) * DH]
                    s = lax.dot_general(
                        q_bh, k_bh, (((1,), (1,)), ((), ())),
                        preferred_element_type=jnp.float32) * SCALE
                    w = jnp.exp(jnp.where(mask, s, NEG))
                    l = jnp.sum(w, axis=1, keepdims=True)
                    o = jnp.dot(w, v_bh, preferred_element_type=jnp.float32)
                    L[c][b][h] = l
                    O[c][b][h] = o
                    pack(c, b, h)
            start_round(0, c)

        for r in range(1, len(ROUNDS)):
            for c in range(NC):
                rdmas[r - 1][c].wait()
                for b in range(B):
                    for h in range(HQ):
                        oj = comm_recv[r - 1, c, b, :, h * DH:(h + 1) * DH]
                        lj = comm_recv[r - 1, c, b, :, DF + h:DF + h + 1]
                        L[c][b][h] = L[c][b][h] + lj.astype(jnp.float32)
                        O[c][b][h] = O[c][b][h] + oj.astype(jnp.float32)
                        pack(c, b, h)
                start_round(r, c)

        last = len(ROUNDS) - 1
        for c in range(NC):
            rdmas[last][c].wait()
            for b in range(B):
                for h in range(HQ):
                    oj = comm_recv[last, c, b, :, h * DH:(h + 1) * DH]
                    lj = comm_recv[last, c, b, :, DF + h:DF + h + 1]
                    L[c][b][h] = L[c][b][h] + lj.astype(jnp.float32)
                    O[c][b][h] = O[c][b][h] + oj.astype(jnp.float32)
                ctx = jnp.concatenate(
                    [O[c][b][h] * (1.0 / L[c][b][h]) for h in range(HQ)],
                    axis=1)
                out_ref[b, c * CH:(c + 1) * CH, :] = jnp.dot(
                    ctx, wo_ref[:, :], preferred_element_type=jnp.float32)

    return pl.pallas_call(
        body,
        out_shape=jax.ShapeDtypeStruct((B, SQ, DMODEL), jnp.float32),
        in_specs=[pl.BlockSpec(memory_space=pltpu.VMEM)] * 5,
        out_specs=pl.BlockSpec(memory_space=pltpu.VMEM),
        scratch_shapes=[
            pltpu.VMEM((NC, B, CH, PACK), jnp.bfloat16),
            pltpu.VMEM((3, NC, B, CH, PACK), jnp.bfloat16),
            pltpu.SemaphoreType.DMA((3, NC)),
            pltpu.SemaphoreType.DMA((3, NC)),
        ],
        compiler_params=pltpu.CompilerParams(collective_id=0),
    )(x, Wq, K2, V2, Wo)


# device time: 49437 ns/iter; 1.2221x vs baseline; 1.2221x over previous
import jax
import jax.numpy as jnp
from jax import lax
from jax.experimental import pallas as pl
from jax.experimental.pallas import tpu as pltpu

N_DEV = 8
B = 2
SQ = 512
DMODEL = 768
HQ = 8
DH = 64
DF = HQ * DH
PACK = DF + 8
SKV_SHARD = 512
SCALE = 0.125
NEG = -1e9
ROUNDS = (1, 3, 4)
NC = 4
CH = SQ // NC


def kernel(x, Wq, K_ext, V_ext, Wo):
    K2 = K_ext.reshape(B, SKV_SHARD, DF)
    V2 = V_ext.reshape(B, SKV_SHARD, DF)

    def body(x_ref, wq_ref, k_ref, v_ref, wo_ref, out_ref,
             comm_send, comm_recv, send_sems, recv_sems):
        my = lax.axis_index("i")

        barrier_sem = pltpu.get_barrier_semaphore()
        for xr in ROUNDS:
            pl.semaphore_signal(
                barrier_sem, inc=1,
                device_id=(my ^ xr,), device_id_type=pl.DeviceIdType.MESH,
            )
        pl.semaphore_wait(barrier_sem, 3)

        kj = lax.broadcasted_iota(jnp.int32, (CH, SKV_SHARD), 1) + my * SKV_SHARD

        L = [[[None] * HQ for _ in range(B)] for _ in range(NC)]
        O = [[[None] * HQ for _ in range(B)] for _ in range(NC)]
        rdmas = [[None] * NC for _ in range(len(ROUNDS))]

        def pack(c, b, h):
            comm_send[c, b, :, h * DH:(h + 1) * DH] = O[c][b][h].astype(jnp.bfloat16)
            comm_send[c, b, :, DF + h:DF + h + 1] = L[c][b][h].astype(jnp.bfloat16)

        def start_round(r, c):
            rdma = pltpu.make_async_remote_copy(
                src_ref=comm_send.at[c],
                dst_ref=comm_recv.at[r, c],
                send_sem=send_sems.at[r, c],
                recv_sem=recv_sems.at[r, c],
                device_id=(my ^ ROUNDS[r],),
                device_id_type=pl.DeviceIdType.MESH,
            )
            rdma.start()
            rdmas[r][c] = rdma

        for c in range(NC):
            qi = lax.broadcasted_iota(jnp.int32, (CH, SKV_SHARD), 0) + c * CH
            mask = (jnp.abs(qi - kj) <= 128) | (kj < 32) | (qi < 32)
            for b in range(B):
                q_bc = jnp.dot(x_ref[b, c * CH:(c + 1) * CH, :], wq_ref[:, :],
                               preferred_element_type=jnp.float32)
                for h in range(HQ):
                    q_bh = q_bc[:, h * DH:(h + 1) * DH]
                    k_bh = k_ref[b, :, h * DH:(h + 1) * DH]
                    v_bh = v_ref[b, :, h * DH:(h + 1) * DH]
                    s = lax.dot_general(
                        q_bh, k_bh, (((1,), (1,)), ((), ())),
                        preferred_element_type=jnp.float32) * SCALE
                    w = jnp.exp(jnp.where(mask, s, NEG))
                    l = jnp.sum(w, axis=1, keepdims=True)
                    o = jnp.dot(w, v_bh, preferred_element_type=jnp.float32)
                    L[c][b][h] = l
                    O[c][b][h] = o
                    pack(c, b, h)
            start_round(0, c)

        for r in range(1, len(ROUNDS)):
            for c in range(NC):
                rdmas[r - 1][c].wait()
                for b in range(B):
                    for h in range(HQ):
                        oj = comm_recv[r - 1, c, b, :, h * DH:(h + 1) * DH]
                        lj = comm_recv[r - 1, c, b, :, DF + h:DF + h + 1]
                        L[c][b][h] = L[c][b][h] + lj.astype(jnp.float32)
                        O[c][b][h] = O[c][b][h] + oj.astype(jnp.float32)
                        pack(c, b, h)
                start_round(r, c)

        last = len(ROUNDS) - 1
        for c in range(NC):
            rdmas[last][c].wait()
            for b in range(B):
                for h in range(HQ):
                    oj = comm_recv[last, c, b, :, h * DH:(h + 1) * DH]
                    lj = comm_recv[last, c, b, :, DF + h:DF + h + 1]
                    L[c][b][h] = L[c][b][h] + lj.astype(jnp.float32)
                    O[c][b][h] = O[c][b][h] + oj.astype(jnp.float32)
                ctx = jnp.concatenate(
                    [O[c][b][h] * (1.0 / L[c][b][h]) for h in range(HQ)],
                    axis=1)
                out_ref[b, c * CH:(c + 1) * CH, :] = jnp.dot(
                    ctx, wo_ref[:, :], preferred_element_type=jnp.float32)

    return pl.pallas_call(
        body,
        out_shape=jax.ShapeDtypeStruct((B, SQ, DMODEL), jnp.float32),
        in_specs=[pl.BlockSpec(memory_space=pltpu.VMEM)] * 5,
        out_specs=pl.BlockSpec(memory_space=pltpu.VMEM),
        scratch_shapes=[
            pltpu.VMEM((NC, B, CH, PACK), jnp.bfloat16),
            pltpu.VMEM((3, NC, B, CH, PACK), jnp.bfloat16),
            pltpu.SemaphoreType.DMA((3, NC)),
            pltpu.SemaphoreType.DMA((3, NC)),
        ],
        compiler_params=pltpu.CompilerParams(collective_id=0),
    )(x, Wq, K2, V2, Wo)


# device time: 47320 ns/iter; 1.2768x vs baseline; 1.0447x over previous
import jax
import jax.numpy as jnp
from jax import lax
from jax.experimental import pallas as pl
from jax.experimental.pallas import tpu as pltpu

N_DEV = 8
B = 2
SQ = 512
DMODEL = 768
HQ = 8
DH = 64
DF = HQ * DH
DHL = DH + 1
PACK = HQ * DHL
SKV_SHARD = 512
SCALE_LOG2E = 0.125 * 1.4426950408889634
NEG = -1e9
ROUNDS = (1, 3, 4)
CHS = (64, 160, 160, 128)
OFFS = (0, 64, 224, 384)
NC = len(CHS)
CHMAX = max(CHS)


def kernel(x, Wq, K_ext, V_ext, Wo):
    K2 = K_ext.reshape(B, SKV_SHARD, DF)
    my_out = lax.axis_index("i")
    qi_g = jnp.arange(SQ, dtype=jnp.int32)[:, None]
    kj_g = jnp.arange(SKV_SHARD, dtype=jnp.int32)[None, :] + my_out * SKV_SHARD
    mask01 = ((jnp.abs(qi_g - kj_g) <= 128) | (kj_g < 32) | (qi_g < 32)
              ).astype(jnp.float32)
    V3 = jnp.concatenate(
        [V_ext, jnp.ones((B, SKV_SHARD, HQ, 1), jnp.float32)], axis=-1
    ).reshape(B, SKV_SHARD, PACK)

    def body(x_ref, wq_ref, k_ref, v_ref, wo_ref, mask_ref, out_ref,
             comm_send, comm_recv, send_sems, recv_sems):
        my = lax.axis_index("i")

        barrier_sem = pltpu.get_barrier_semaphore()
        for xr in ROUNDS:
            pl.semaphore_signal(
                barrier_sem, inc=1,
                device_id=(my ^ xr,), device_id_type=pl.DeviceIdType.MESH,
            )
        pl.semaphore_wait(barrier_sem, 3)

        rdmas = [[None] * NC for _ in range(len(ROUNDS))]
        take0 = my <= 1
        take1 = (my == 2) | (my == 3)
        take2 = my >= 4
        takes = (take0, take1, take2)

        def start_round(r, c):
            ch = CHS[c]
            rdma = pltpu.make_async_remote_copy(
                src_ref=comm_send.at[c, :, 0:ch, :],
                dst_ref=comm_recv.at[r, c, :, 0:ch, :],
                send_sem=send_sems.at[r, c],
                recv_sem=recv_sems.at[r, c],
                device_id=(my ^ ROUNDS[r],),
                device_id_type=pl.DeviceIdType.MESH,
            )
            rdma.start()
            rdmas[r][c] = rdma

        for c in range(NC):
            ch, off = CHS[c], OFFS[c]
            mask = mask_ref[off:off + ch, :]
            for b in range(B):
                q_bc = jnp.dot(x_ref[b, off:off + ch, :], wq_ref[:, :],
                               preferred_element_type=jnp.float32)
                os = []
                for h in range(HQ):
                    q_bh = q_bc[:, h * DH:(h + 1) * DH]
                    k_bh = k_ref[b, :, h * DH:(h + 1) * DH]
                    v_bh = v_ref[b, :, h * DHL:(h + 1) * DHL]
                    s = lax.dot_general(
                        q_bh, k_bh, (((1,), (1,)), ((), ())),
                        preferred_element_type=jnp.float32) * SCALE_LOG2E
                    w = jnp.exp2(s) * mask
                    os.append(jnp.dot(w, v_bh,
                                      preferred_element_type=jnp.float32))
                packed = jnp.concatenate(os, axis=1)
                comm_send[c, b, 0:ch, :] = packed.astype(jnp.bfloat16)
            start_round(0, c)

        for r in range(1, len(ROUNDS)):
            for c in range(NC):
                ch = CHS[c]

                def _take(c=c, ch=ch, r=r):
                    rdmas[r - 1][c].wait()
                    comm_send[c, :, 0:ch, :] = (
                        comm_send[c, :, 0:ch, :]
                        + comm_recv[r - 1, c, :, 0:ch, :])

                if c == 0:
                    _take()
                else:
                    pl.when(takes[r - 1])(_take)
                start_round(r, c)

        last = len(ROUNDS) - 1
        for c in range(NC):
            ch, off = CHS[c], OFFS[c]
            if c == 0:
                rdmas[last][c].wait()
            else:
                pl.when(takes[last])(lambda c=c: rdmas[last][c].wait())
            for b in range(B):
                base = comm_send[c, b, 0:ch, :].astype(jnp.float32)
                if c == 0:
                    tot = base + comm_recv[last, c, b, 0:ch, :].astype(
                        jnp.float32)
                else:
                    tot = base + jnp.where(
                        takes[last],
                        comm_recv[last, c, b, 0:ch, :].astype(jnp.float32),
                        0.0)
                ctx = jnp.concatenate(
                    [tot[:, h * DHL:h * DHL + DH]
                     * (1.0 / tot[:, h * DHL + DH:(h + 1) * DHL])
                     for h in range(HQ)], axis=1)
                out_ref[b, off:off + ch, :] = jnp.dot(
                    ctx, wo_ref[:, :], preferred_element_type=jnp.float32)

        for r in range(len(ROUNDS)):
            for c in range(1, NC):
                pl.when(jnp.logical_not(takes[r]))(
                    lambda r=r, c=c: rdmas[r][c].wait())

    return pl.pallas_call(
        body,
        out_shape=jax.ShapeDtypeStruct((B, SQ, DMODEL), jnp.float32),
        in_specs=[pl.BlockSpec(memory_space=pltpu.VMEM)] * 6,
        out_specs=pl.BlockSpec(memory_space=pltpu.VMEM),
        scratch_shapes=[
            pltpu.VMEM((NC, B, CHMAX, PACK), jnp.bfloat16),
            pltpu.VMEM((3, NC, B, CHMAX, PACK), jnp.bfloat16),
            pltpu.SemaphoreType.DMA((3, NC)),
            pltpu.SemaphoreType.DMA((3, NC)),
        ],
        compiler_params=pltpu.CompilerParams(collective_id=0),
    )(x, Wq, K2, V3, Wo, mask01)
